# Initial kernel scaffold; baseline (speedup 1.0000x reference)
import functools

import jax
import jax.numpy as jnp
from jax import lax
from jax.experimental import pallas as pl
from jax.experimental.pallas import tpu as pltpu
from jax.experimental.pallas import tpu_sc as plsc

N = 10000
E = 320000
D = 128
C = 64
NC = 2
NS = 16
NW = NC * NS
CHUNK = 128
NCH = 79
EPT = NCH * CHUNK
EPAD = NW * EPT
NPAD = 10016
RPT = NPAD // NS

_MESH = plsc.VectorSubcoreMesh(
    core_axis_name="c", subcore_axis_name="s", num_cores=NC, num_subcores=NS
)


def _zero_fill(buf, rows, width):
    zeros16 = jnp.zeros((16,), jnp.float32)

    @pl.loop(0, rows)
    def _(r):
        @pl.loop(0, width // 16)
        def _(q):
            buf[r, pl.ds(q * 16, 16)] = zeros16


def _init_acc_rows(rows_v, acc_sh, base):
    nfull = RPT // CHUNK
    rem = RPT - nfull * CHUNK
    for k in range(nfull):
        pltpu.sync_copy(rows_v, acc_sh.at[pl.ds(base + k * CHUNK, CHUNK)])
    pltpu.sync_copy(rows_v.at[pl.ds(0, rem)],
                    acc_sh.at[pl.ds(base + nfull * CHUNK, rem)])


@functools.partial(
    pl.kernel,
    out_type=jax.ShapeDtypeStruct((NC, NPAD, 16), jnp.float32),
    mesh=_MESH,
    scratch_types=[
        pltpu.VMEM((NCH, CHUNK), jnp.int32),
        pltpu.VMEM((CHUNK, 16), jnp.float32),
        pltpu.VMEM_SHARED((NPAD, 16), jnp.float32),
    ],
)
def _hist(dstr_hbm, out_hbm, dst_v, ones_v, acc_sh):
    core = lax.axis_index("c")
    sid = lax.axis_index("s")
    wid = core * NS + sid
    base = sid * RPT
    pltpu.sync_copy(dstr_hbm.at[wid], dst_v)
    _zero_fill(ones_v, CHUNK, 16)
    _init_acc_rows(ones_v, acc_sh, base)
    ones16 = jnp.ones((16,), jnp.float32)

    @pl.loop(0, CHUNK)
    def _(r):
        ones_v[r, pl.ds(0, 16)] = ones16

    plsc.subcore_barrier()

    @pl.loop(0, NCH)
    def _(j):
        pltpu.sync_copy(ones_v, acc_sh.at[dst_v.at[j]], add=True)

    plsc.subcore_barrier()
    pltpu.sync_copy(acc_sh.at[pl.ds(base, RPT)],
                    out_hbm.at[core].at[pl.ds(base, RPT)])


@functools.partial(
    pl.kernel,
    out_type=jax.ShapeDtypeStruct((NC, NPAD, C), jnp.float32),
    mesh=_MESH,
    scratch_types=[
        pltpu.VMEM((NCH, CHUNK), jnp.int32),
        pltpu.VMEM((NCH, CHUNK), jnp.int32),
        pltpu.VMEM((CHUNK, C), jnp.float32),
        pltpu.VMEM_SHARED((NPAD, C), jnp.float32),
    ],
)
def _hop(g_hbm, srcr_hbm, dstr_hbm, out_hbm, src_v, dst_v, rows_v, acc_sh):
    core = lax.axis_index("c")
    sid = lax.axis_index("s")
    wid = core * NS + sid
    base = sid * RPT
    pltpu.sync_copy(srcr_hbm.at[wid], src_v)
    pltpu.sync_copy(dstr_hbm.at[wid], dst_v)
    _zero_fill(rows_v, CHUNK, C)
    _init_acc_rows(rows_v, acc_sh, base)
    plsc.subcore_barrier()

    @pl.loop(0, NCH)
    def _(j):
        pltpu.sync_copy(g_hbm.at[src_v.at[j]], rows_v)
        pltpu.sync_copy(rows_v, acc_sh.at[dst_v.at[j]], add=True)

    plsc.subcore_barrier()
    pltpu.sync_copy(acc_sh.at[pl.ds(base, RPT)],
                    out_hbm.at[core].at[pl.ds(base, RPT)])


def _deg_cols(hist_ref):
    cnt = hist_ref[0, :N, 0:1] + hist_ref[1, :N, 0:1]
    return cnt + 1.0


def _prep_body(x_ref, wt_ref, hist_ref, g1_ref):
    z = lax.dot_general(x_ref[...], wt_ref[...], (((1,), (0,)), ((), ())),
                        precision=lax.Precision.HIGHEST,
                        preferred_element_type=jnp.float32)
    s = lax.rsqrt(_deg_cols(hist_ref))
    g1_ref[...] = s * z


def _mid_body(acc_ref, g1_ref, hist_ref, g2_ref):
    acc = acc_ref[0, :N, :] + acc_ref[1, :N, :] + g1_ref[...]
    g2_ref[...] = acc / _deg_cols(hist_ref)


def _final_body(acc_ref, g2_ref, hist_ref, b_ref, out_ref):
    acc = acc_ref[0, :N, :] + acc_ref[1, :N, :] + g2_ref[...]
    logits = lax.rsqrt(_deg_cols(hist_ref)) * acc + b_ref[...]
    m = jnp.max(logits, axis=1, keepdims=True)
    shifted = logits - m
    lse = jnp.log(jnp.sum(jnp.exp(shifted), axis=1, keepdims=True))
    out_ref[...] = shifted - lse


def kernel(x, edge_index, W, b):
    src = edge_index[0]
    dst = edge_index[1]
    pad = EPAD - E
    srcr = jnp.concatenate([src, jnp.zeros((pad,), jnp.int32)]).reshape(
        NW, NCH, CHUNK)
    dstr = jnp.concatenate([dst, jnp.full((pad,), N, jnp.int32)]).reshape(
        NW, NCH, CHUNK)

    hist = _hist(dstr)

    g1 = pl.pallas_call(
        _prep_body,
        out_shape=jax.ShapeDtypeStruct((N, C), jnp.float32),
    )(x, W.T, hist)

    acc1 = _hop(g1, srcr, dstr)

    g2 = pl.pallas_call(
        _mid_body,
        out_shape=jax.ShapeDtypeStruct((N, C), jnp.float32),
    )(acc1, g1, hist)

    acc2 = _hop(g2, srcr, dstr)

    out = pl.pallas_call(
        _final_body,
        out_shape=jax.ShapeDtypeStruct((N, C), jnp.float32),
    )(acc2, g2, hist, b.reshape(1, C))

    return out


# speedup vs baseline: 20.2403x; 20.2403x over previous
import functools

import jax
import jax.numpy as jnp
from jax import lax
from jax.experimental import pallas as pl
from jax.experimental.pallas import tpu as pltpu
from jax.experimental.pallas import tpu_sc as plsc

N = 10000
E = 320000
D = 128
C = 64
NC = 2
NS = 16
NW = NC * NS
CHUNK = 128
NCH = 79
EPT = NCH * CHUNK
EPAD = NW * EPT
NPAD = 10112
RPT = NPAD // NS

_MESH = plsc.VectorSubcoreMesh(
    core_axis_name="c", subcore_axis_name="s", num_cores=NC, num_subcores=NS
)
_SC_PARAMS = pltpu.CompilerParams(use_tc_tiling_on_sc=False)


def _zero_fill(buf, rows, width):
    zeros16 = jnp.zeros((16,), jnp.float32)

    @pl.loop(0, rows)
    def _(r):
        @pl.loop(0, width // 16)
        def _(q):
            buf[r, pl.ds(q * 16, 16)] = zeros16


def _init_acc_rows(rows_v, acc_sh, base):
    nfull = RPT // CHUNK
    rem = RPT - nfull * CHUNK
    for k in range(nfull):
        pltpu.sync_copy(rows_v, acc_sh.at[pl.ds(base + k * CHUNK, CHUNK)])
    pltpu.sync_copy(rows_v.at[pl.ds(0, rem)],
                    acc_sh.at[pl.ds(base + nfull * CHUNK, rem)])


@functools.partial(
    pl.kernel,
    out_type=jax.ShapeDtypeStruct((NC, NPAD, 16), jnp.float32),
    mesh=_MESH,
    scratch_types=[
        pltpu.VMEM((NCH, CHUNK), jnp.int32),
        pltpu.VMEM((CHUNK, 16), jnp.float32),
        pltpu.VMEM_SHARED((NPAD, 16), jnp.float32),
    ],
    compiler_params=_SC_PARAMS,
)
def _hist(dstr_hbm, out_hbm, dst_v, ones_v, acc_sh):
    core = lax.axis_index("c")
    sid = lax.axis_index("s")
    wid = core * NS + sid
    base = sid * RPT
    pltpu.sync_copy(dstr_hbm.at[wid], dst_v)
    _zero_fill(ones_v, CHUNK, 16)
    _init_acc_rows(ones_v, acc_sh, base)
    ones16 = jnp.ones((16,), jnp.float32)

    @pl.loop(0, CHUNK)
    def _(r):
        ones_v[r, pl.ds(0, 16)] = ones16

    plsc.subcore_barrier()

    @pl.loop(0, NCH)
    def _(j):
        pltpu.sync_copy(ones_v, acc_sh.at[dst_v.at[j]], add=True)

    plsc.subcore_barrier()
    pltpu.sync_copy(acc_sh.at[pl.ds(base, RPT)],
                    out_hbm.at[core].at[pl.ds(base, RPT)])


@functools.partial(
    pl.kernel,
    out_type=jax.ShapeDtypeStruct((NC, NPAD, C), jnp.float32),
    mesh=_MESH,
    scratch_types=[
        pltpu.VMEM((NCH, CHUNK), jnp.int32),
        pltpu.VMEM((NCH, CHUNK), jnp.int32),
        pltpu.VMEM((CHUNK, C), jnp.float32),
        pltpu.VMEM_SHARED((NPAD, C), jnp.float32),
    ],
    compiler_params=_SC_PARAMS,
)
def _hop(g_hbm, srcr_hbm, dstr_hbm, out_hbm, src_v, dst_v, rows_v, acc_sh):
    core = lax.axis_index("c")
    sid = lax.axis_index("s")
    wid = core * NS + sid
    base = sid * RPT
    pltpu.sync_copy(srcr_hbm.at[wid], src_v)
    pltpu.sync_copy(dstr_hbm.at[wid], dst_v)
    _zero_fill(rows_v, CHUNK, C)
    _init_acc_rows(rows_v, acc_sh, base)
    plsc.subcore_barrier()

    @pl.loop(0, NCH)
    def _(j):
        pltpu.sync_copy(g_hbm.at[src_v.at[j]], rows_v)
        pltpu.sync_copy(rows_v, acc_sh.at[dst_v.at[j]], add=True)

    plsc.subcore_barrier()
    pltpu.sync_copy(acc_sh.at[pl.ds(base, RPT)],
                    out_hbm.at[core].at[pl.ds(base, RPT)])


def _deg_cols(hist_ref):
    cnt = hist_ref[0, :N, 0:1] + hist_ref[1, :N, 0:1]
    return cnt + 1.0


def _prep_body(x_ref, wt_ref, hist_ref, g1_ref):
    z = lax.dot_general(x_ref[...], wt_ref[...], (((1,), (0,)), ((), ())),
                        precision=lax.Precision.HIGHEST,
                        preferred_element_type=jnp.float32)
    s = lax.rsqrt(_deg_cols(hist_ref))
    g1_ref[...] = s * z


def _mid_body(acc_ref, g1_ref, hist_ref, g2_ref):
    acc = acc_ref[0, :N, :] + acc_ref[1, :N, :] + g1_ref[...]
    g2_ref[...] = acc / _deg_cols(hist_ref)


def _final_body(acc_ref, g2_ref, hist_ref, b_ref, out_ref):
    acc = acc_ref[0, :N, :] + acc_ref[1, :N, :] + g2_ref[...]
    logits = lax.rsqrt(_deg_cols(hist_ref)) * acc + b_ref[...]
    m = jnp.max(logits, axis=1, keepdims=True)
    shifted = logits - m
    lse = jnp.log(jnp.sum(jnp.exp(shifted), axis=1, keepdims=True))
    out_ref[...] = shifted - lse


def kernel(x, edge_index, W, b):
    src = edge_index[0]
    dst = edge_index[1]
    pad = EPAD - E
    srcr = jnp.concatenate([src, jnp.zeros((pad,), jnp.int32)]).reshape(
        NW, NCH, CHUNK)
    dstr = jnp.concatenate([dst, jnp.full((pad,), N, jnp.int32)]).reshape(
        NW, NCH, CHUNK)

    hist = _hist(dstr)

    g1 = pl.pallas_call(
        _prep_body,
        out_shape=jax.ShapeDtypeStruct((N, C), jnp.float32),
    )(x, W.T, hist)

    acc1 = _hop(g1, srcr, dstr)

    g2 = pl.pallas_call(
        _mid_body,
        out_shape=jax.ShapeDtypeStruct((N, C), jnp.float32),
    )(acc1, g1, hist)

    acc2 = _hop(g2, srcr, dstr)

    out = pl.pallas_call(
        _final_body,
        out_shape=jax.ShapeDtypeStruct((N, C), jnp.float32),
    )(acc2, g2, hist, b.reshape(1, C))

    return out
